# Initial kernel scaffold; baseline (speedup 1.0000x reference)
import jax
import jax.numpy as jnp
from jax.experimental import pallas as pl


def kernel(nodes, samp_neighs, val_lens, feats_data, W0, W1):
    raise NotImplementedError("write your pallas kernel here")


# speedup vs baseline: 1.4178x; 1.4178x over previous
import functools

import jax
import jax.numpy as jnp
from jax import lax
from jax.experimental import pallas as pl
from jax.experimental.pallas import tpu as pltpu
from jax.experimental.pallas import tpu_sc as plsc

N_TABLE = 100000
NB = 10000
S = 32
D = 128

NW = 32
NPAD = 10240
PW = NPAD // NW
CH = 128
NCH = PW * S // CH
AGG_ROWS = 336
TRASH = 320
SELF_CH = 64
NSELF = PW // SELF_CH


def _sc_gather_agg(feats, samp_flat, nodes_pad, lens_exp, zrows):
    mesh = plsc.VectorSubcoreMesh(core_axis_name="c", subcore_axis_name="s")

    @functools.partial(
        pl.kernel,
        out_type=(
            jax.ShapeDtypeStruct((NPAD, D), jnp.float32),
            jax.ShapeDtypeStruct((NPAD, D), jnp.float32),
        ),
        mesh=mesh,
        scratch_types=[
            pltpu.VMEM((PW * S,), jnp.int32),
            pltpu.VMEM((NCH, CH), jnp.int32),
            pltpu.VMEM((PW,), jnp.int32),
            pltpu.VMEM((PW * S,), jnp.int32),
            pltpu.VMEM((CH, D), jnp.float32),
            pltpu.VMEM((CH, D), jnp.float32),
            pltpu.VMEM((SELF_CH, D), jnp.float32),
            pltpu.VMEM_SHARED((16 * AGG_ROWS, D), jnp.float32),
            pltpu.SemaphoreType.DMA,
            pltpu.SemaphoreType.DMA,
        ],
    )
    def k(feats_h, samp_h, nodes_h, lensx_h, z_h, agg_out, self_out,
          samp_v, sidx_v, nodes_v, lensx_v, rows0, rows1, sbuf, agg_sh,
          gsem0, gsem1):
        cid = lax.axis_index("c")
        sid = lax.axis_index("s")
        wid = sid * 2 + cid
        base = sid * AGG_ROWS

        pltpu.sync_copy(samp_h.at[pl.ds(wid * PW * S, PW * S)], samp_v)
        pltpu.sync_copy(nodes_h.at[pl.ds(wid * PW, PW)], nodes_v)
        pltpu.sync_copy(lensx_h.at[pl.ds(wid * PW * S, PW * S)], lensx_v)

        @pl.loop(0, AGG_ROWS // 16)
        def _zero(t):
            pltpu.sync_copy(z_h, agg_sh.at[pl.ds(base + t * 16, 16)])

        @pl.loop(0, NCH)
        def _mk(c):
            for kk in range(CH // 16):
                p0 = c * CH + kk * 16
                p = p0 + lax.iota(jnp.int32, 16)
                i = lax.shift_right_logical(p, 5)
                j = jnp.bitwise_and(p, S - 1)
                lens16 = lensx_v[pl.ds(p0, 16)]
                val = jnp.where(j < lens16, i, TRASH) + base
                sidx_v[c, pl.ds(kk * 16, 16)] = val

        @pl.loop(0, NSELF)
        def _self(t):
            pltpu.sync_copy(feats_h.at[nodes_v.at[pl.ds(t * SELF_CH, SELF_CH)]],
                            sbuf)
            pltpu.sync_copy(sbuf, self_out.at[pl.ds(wid * PW + t * SELF_CH,
                                                    SELF_CH)])

        pltpu.async_copy(feats_h.at[samp_v.at[pl.ds(0, CH)]], rows0, gsem0)

        @pl.loop(0, NCH // 2)
        def _main(t):
            c0 = 2 * t
            c1 = c0 + 1
            pltpu.async_copy(feats_h.at[samp_v.at[pl.ds(c1 * CH, CH)]],
                             rows1, gsem1)
            pltpu.make_async_copy(feats_h.at[samp_v.at[pl.ds(c0 * CH, CH)]],
                                  rows0, gsem0).wait()
            pltpu.sync_copy(rows0, agg_sh.at[sidx_v.at[c0]], add=True)

            @pl.when(t < NCH // 2 - 1)
            def _():
                pltpu.async_copy(feats_h.at[samp_v.at[pl.ds((c0 + 2) * CH, CH)]],
                                 rows0, gsem0)

            pltpu.make_async_copy(feats_h.at[samp_v.at[pl.ds(c1 * CH, CH)]],
                                  rows1, gsem1).wait()
            pltpu.sync_copy(rows1, agg_sh.at[sidx_v.at[c1]], add=True)

        @pl.loop(0, PW // SELF_CH)
        def _out(t):
            pltpu.sync_copy(agg_sh.at[pl.ds(base + t * SELF_CH, SELF_CH)],
                            agg_out.at[pl.ds(wid * PW + t * SELF_CH, SELF_CH)])

    return k(feats, samp_flat, nodes_pad, lens_exp, zrows)


def _tc_dense(self_e, agg_sum, lensf, w1a, w1b):
    BLK = 512

    def body(self_ref, agg_ref, lens_ref, wa_ref, wb_ref, out_ref):
        recip = 1.0 / jnp.maximum(lens_ref[...], 1.0)
        h_self = lax.dot_general(self_ref[...], wa_ref[...],
                                 (((1,), (1,)), ((), ())),
                                 preferred_element_type=jnp.float32)
        h_agg = lax.dot_general(agg_ref[...], wb_ref[...],
                                (((1,), (1,)), ((), ())),
                                preferred_element_type=jnp.float32)
        out_ref[...] = jnp.maximum(h_self + recip * h_agg, 0.0)

    return pl.pallas_call(
        body,
        grid=(NPAD // BLK,),
        in_specs=[
            pl.BlockSpec((BLK, D), lambda i: (i, 0)),
            pl.BlockSpec((BLK, D), lambda i: (i, 0)),
            pl.BlockSpec((BLK, 1), lambda i: (i, 0)),
            pl.BlockSpec((D, D), lambda i: (0, 0)),
            pl.BlockSpec((D, D), lambda i: (0, 0)),
        ],
        out_specs=pl.BlockSpec((BLK, D), lambda i: (i, 0)),
        out_shape=jax.ShapeDtypeStruct((NPAD, D), jnp.float32),
    )(self_e, agg_sum, lensf, w1a, w1b)


def kernel(nodes, samp_neighs, val_lens, feats_data, W0, W1):
    del W0
    nodes_pad = jnp.pad(nodes.astype(jnp.int32), (0, NPAD - NB))
    samp_pad = jnp.pad(samp_neighs.astype(jnp.int32),
                       ((0, NPAD - NB), (0, 0))).reshape(-1)
    lens_pad = jnp.pad(val_lens.astype(jnp.int32), (0, NPAD - NB))
    zrows = jnp.zeros((16, D), jnp.float32)
    lens_exp = jnp.repeat(lens_pad, S)
    agg_sum, self_e = _sc_gather_agg(feats_data, samp_pad, nodes_pad,
                                     lens_exp, zrows)
    lensf = lens_pad.astype(jnp.float32).reshape(NPAD, 1)
    out = _tc_dense(self_e, agg_sum, lensf, W1[:, :D], W1[:, D:])
    return out[:NB]
